# baseline (speedup 1.0000x reference)
import jax
import jax.numpy as jnp
from jax.experimental import pallas as pl
from jax.experimental.pallas import tpu as pltpu


def _add_pos_kernel(x_ref, pos_ref, out_ref):
    out_ref[...] = x_ref[...] + pos_ref[...][None, :, :]


def kernel(x, pos_table):
    batch, seq, d_model = x.shape
    bs = 2048
    bh = 1
    grid = (seq // bs, batch)
    return pl.pallas_call(
        _add_pos_kernel,
        grid=grid,
        in_specs=[
            pl.BlockSpec((bh, bs, d_model), lambda i, j: (j, i, 0)),
            pl.BlockSpec((bs, d_model), lambda i, j: (i, 0)),
        ],
        out_specs=pl.BlockSpec((bh, bs, d_model), lambda i, j: (j, i, 0)),
        out_shape=jax.ShapeDtypeStruct((batch, seq, d_model), x.dtype),
        compiler_params=pltpu.CompilerParams(
            dimension_semantics=("parallel", "parallel"),
        ),
    )(x, pos_table[:seq])


# speedup vs baseline: 1.0101x; 1.0101x over previous
import jax
import jax.numpy as jnp
from jax.experimental import pallas as pl
from jax.experimental.pallas import tpu as pltpu


def _add_pos_kernel(x_ref, pos_ref, out_ref):
    out_ref[...] = x_ref[...] + pos_ref[...][None, :, :]


def kernel(x, pos_table):
    batch, seq, d_model = x.shape
    bs = 1024
    bh = batch // 2
    grid = (seq // bs, batch // bh)
    return pl.pallas_call(
        _add_pos_kernel,
        grid=grid,
        in_specs=[
            pl.BlockSpec((bh, bs, d_model), lambda i, j: (j, i, 0)),
            pl.BlockSpec((bs, d_model), lambda i, j: (i, 0)),
        ],
        out_specs=pl.BlockSpec((bh, bs, d_model), lambda i, j: (j, i, 0)),
        out_shape=jax.ShapeDtypeStruct((batch, seq, d_model), x.dtype),
        compiler_params=pltpu.CompilerParams(
            dimension_semantics=("parallel", "parallel"),
        ),
    )(x, pos_table[:seq])
